# Initial kernel scaffold; baseline (speedup 1.0000x reference)
import jax
import jax.numpy as jnp
from jax.experimental import pallas as pl


def kernel(links, hw_tab, wk_tab, dt_tab, tm_tab, W1, b1, W2, b2, args):
    raise NotImplementedError("write your pallas kernel here")


# speedup vs baseline: 3.0529x; 3.0529x over previous
import functools

import jax
import jax.numpy as jnp
from jax import lax
from jax.experimental import pallas as pl
from jax.experimental.pallas import tpu as pltpu
from jax.experimental.pallas import tpu_sc as plsc

_NW = 32
_CH = 512
_SUB = 128
_LANES = 16
_RB = 2048


def _sc_gather_call(fdi, fti, fd_tab, ft_tab):
    n = fdi.shape[0]
    dfd = fd_tab.shape[1]
    dft = ft_tab.shape[1]
    rows_w = n // _NW
    n_chunks = rows_w // _CH
    mesh = plsc.VectorSubcoreMesh(core_axis_name="c", subcore_axis_name="s")

    @functools.partial(
        pl.kernel,
        out_type=(
            jax.ShapeDtypeStruct((n, dfd), jnp.float32),
            jax.ShapeDtypeStruct((n, dft), jnp.float32),
        ),
        mesh=mesh,
        compiler_params=pltpu.CompilerParams(use_tc_tiling_on_sc=False),
        scratch_types=[
            pltpu.VMEM((_SUB,), jnp.int32),
            pltpu.VMEM((_SUB,), jnp.int32),
            pltpu.VMEM((_SUB, dfd), jnp.float32),
            pltpu.VMEM((_SUB, dft), jnp.float32),
            pltpu.SemaphoreType.DMA,
        ],
    )
    def sc_kernel(fdi_h, fti_h, fd_h, ft_h, x1_h, x2_h,
                  fdi_v, fti_v, b1_v, b2_v, sem):
        wid = lax.axis_index("s") * 2 + lax.axis_index("c")
        base = wid * rows_w

        def chunk(ci, carry):
            b = base + ci * _SUB
            pltpu.sync_copy(fdi_h.at[pl.ds(b, _SUB)], fdi_v)
            pltpu.sync_copy(fti_h.at[pl.ds(b, _SUB)], fti_v)

            c1 = pltpu.async_copy(fd_h.at[fdi_v], b1_v, sem)
            c2 = pltpu.async_copy(ft_h.at[fti_v], b2_v, sem)
            c1.wait()
            c2.wait()

            pltpu.sync_copy(b1_v, x1_h.at[pl.ds(b, _SUB), :])
            pltpu.sync_copy(b2_v, x2_h.at[pl.ds(b, _SUB), :])
            return carry

        lax.fori_loop(0, rows_w // _SUB, chunk, 0)

    return sc_kernel(fdi, fti, fd_tab, ft_tab)


def _tc_body(lk_ref, x1_ref, x2_ref, w1a_ref, w1b_ref, w2_ref, b1_ref, b2_ref,
             feat_ref, wk_ref, dr_ref, tr_ref):
    x1 = x1_ref[...]
    x2 = x2_ref[...]
    h1 = (jnp.dot(x1, w1a_ref[...], preferred_element_type=jnp.float32)
          + jnp.dot(x2, w1b_ref[...], preferred_element_type=jnp.float32)
          + b1_ref[...])
    a = jnp.maximum(h1, 0.01 * h1)
    h2 = jnp.dot(a, w2_ref[...], preferred_element_type=jnp.float32) + b2_ref[...]
    feat_ref[:, 0:2] = lk_ref[:, 1:3]
    feat_ref[:, 2:7] = x2[:, 20:25]
    feat_ref[:, 7:20] = h2[:, 0:13] + x1[:, 0:13]
    feat_ref[:, 20:40] = h2[:, 13:33] + x2[:, 0:20]
    wk_ref[...] = x1[:, 0:3]
    dr_ref[...] = x1[:, 3:13]
    tr_ref[...] = x2[:, 0:20]


def _tc_call(lk2, x1, x2, w1a, w1b, w2t, b1r, b2r):
    n = lk2.shape[0]
    grid = (n // _RB,)
    row_spec = lambda d: pl.BlockSpec((_RB, d), lambda i: (i, 0))
    full = lambda a: pl.BlockSpec(a.shape, lambda i: (0, 0))
    return pl.pallas_call(
        _tc_body,
        grid=grid,
        in_specs=[
            row_spec(lk2.shape[1]),
            row_spec(x1.shape[1]),
            row_spec(x2.shape[1]),
            full(w1a), full(w1b), full(w2t), full(b1r), full(b2r),
        ],
        out_specs=[row_spec(40), row_spec(3), row_spec(10), row_spec(20)],
        out_shape=[
            jax.ShapeDtypeStruct((n, 40), jnp.float32),
            jax.ShapeDtypeStruct((n, 3), jnp.float32),
            jax.ShapeDtypeStruct((n, 10), jnp.float32),
            jax.ShapeDtypeStruct((n, 20), jnp.float32),
        ],
    )(lk2, x1, x2, w1a, w1b, w2t, b1r, b2r)


def kernel(links, hw_tab, wk_tab, dt_tab, tm_tab, W1, b1, W2, b2, args):
    bb, ll, _ = links.shape
    n = bb * ll
    lk2 = links.reshape(n, 6)
    i0 = lk2[:, 0].astype(jnp.int32)
    i3 = lk2[:, 3].astype(jnp.int32)
    i4 = lk2[:, 4].astype(jnp.int32)
    i5 = lk2[:, 5].astype(jnp.int32)

    hw_z = hw_tab.at[0].set(0.0)
    nw, nd = wk_tab.shape[0], dt_tab.shape[0]
    nh, nt = hw_tab.shape[0], tm_tab.shape[0]
    fd_tab = jnp.concatenate([
        jnp.broadcast_to(wk_tab[:, None, :], (nw, nd, 3)),
        jnp.broadcast_to(dt_tab[None, :, :], (nw, nd, 10)),
        jnp.zeros((nw, nd, 3), jnp.float32),
    ], axis=-1).reshape(nw * nd, 16)
    ft_tab = jnp.concatenate([
        jnp.broadcast_to(tm_tab[None, :, :], (nh, nt, 20)),
        jnp.broadcast_to(hw_z[:, None, :], (nh, nt, 5)),
        jnp.zeros((nh, nt, 7), jnp.float32),
    ], axis=-1).reshape(nh * nt, 32)

    fdi = i3 * 367 + i4
    fti = i0 * 1441 + i5
    x1, x2 = _sc_gather_call(fdi, fti, fd_tab, ft_tab)

    w1t = W1.T
    w1a = jnp.concatenate([w1t[0:13, :], jnp.zeros((3, 33), jnp.float32)], 0)
    w1b = jnp.concatenate([w1t[13:33, :], jnp.zeros((12, 33), jnp.float32)], 0)
    feat, wk, dr, tr = _tc_call(lk2, x1, x2, w1a, w1b, W2.T,
                                b1.reshape(1, 33), b2.reshape(1, 33))
    return (feat.reshape(bb, ll, 40),
            (wk.reshape(bb, ll, 3), dr.reshape(bb, ll, 10),
             tr.reshape(bb, ll, 20)))


# speedup vs baseline: 7.6808x; 2.5159x over previous
import functools

import jax
import jax.numpy as jnp
from jax import lax
from jax.experimental import pallas as pl
from jax.experimental.pallas import tpu as pltpu
from jax.experimental.pallas import tpu_sc as plsc

_NW = 32
_SUB = 128
_LANES = 16
_BLC = 256


def _sc_gather_call(fdi, fti, fd_tab, ft_tab):
    n = fdi.shape[0]
    nch = n // _SUB
    rows_w = n // _NW
    mesh = plsc.VectorSubcoreMesh(core_axis_name="c", subcore_axis_name="s")

    @functools.partial(
        pl.kernel,
        out_type=(
            jax.ShapeDtypeStruct((nch, 16, _SUB), jnp.float32),
            jax.ShapeDtypeStruct((nch, 32, _SUB), jnp.float32),
        ),
        mesh=mesh,
        compiler_params=pltpu.CompilerParams(use_tc_tiling_on_sc=False,
                                             needs_layout_passes=False),
        scratch_types=[
            pltpu.VMEM((_SUB,), jnp.int32),
            pltpu.VMEM((_SUB,), jnp.int32),
            pltpu.VMEM((_SUB, 16), jnp.float32),
            pltpu.VMEM((_SUB, 32), jnp.float32),
            pltpu.VMEM((16, _SUB), jnp.float32),
            pltpu.VMEM((32, _SUB), jnp.float32),
            pltpu.SemaphoreType.DMA,
        ],
    )
    def sc_kernel(fdi_h, fti_h, fd_h, ft_h, x1_h, x2_h,
                  fdi_v, fti_v, b1_v, b2_v, t1_v, t2_v, sem):
        iota = lax.iota(jnp.int32, _LANES)
        wid = lax.axis_index("s") * 2 + lax.axis_index("c")
        ch_w = rows_w // _SUB
        base_c = wid * ch_w

        def chunk(ci, carry):
            c = base_c + ci
            b = c * _SUB
            pltpu.sync_copy(fdi_h.at[pl.ds(b, _SUB)], fdi_v)
            pltpu.sync_copy(fti_h.at[pl.ds(b, _SUB)], fti_v)

            c1 = pltpu.async_copy(fd_h.at[fdi_v], b1_v, sem)
            c2 = pltpu.async_copy(ft_h.at[fti_v], b2_v, sem)
            c1.wait()
            c2.wait()

            for f in range(16):
                cols = jnp.full((_LANES,), f, jnp.int32)
                for g in range(_SUB // _LANES):
                    rows = iota + (g * _LANES)
                    v = plsc.load_gather(b1_v, [rows, cols])
                    t1_v[f, pl.ds(g * _LANES, _LANES)] = v
            for f in range(32):
                cols = jnp.full((_LANES,), f, jnp.int32)
                for g in range(_SUB // _LANES):
                    rows = iota + (g * _LANES)
                    v = plsc.load_gather(b2_v, [rows, cols])
                    t2_v[f, pl.ds(g * _LANES, _LANES)] = v

            pltpu.sync_copy(t1_v, x1_h.at[c])
            pltpu.sync_copy(t2_v, x2_h.at[c])
            return carry

        lax.fori_loop(0, ch_w, chunk, 0)

    return sc_kernel(fdi, fti, fd_tab, ft_tab)


def _tc_body(x1_ref, x2_ref, f1_ref, f2_ref, a1_ref, a2_ref, w2_ref,
             b1_ref, b2_ref, feat_ref, wk_ref, dr_ref, tr_ref):
    r = _BLC * _SUB
    x1 = x1_ref[...].transpose(1, 0, 2).reshape(16, r)
    x2 = x2_ref[...].transpose(1, 0, 2).reshape(32, r)
    h1 = (jnp.dot(a1_ref[...], x1, preferred_element_type=jnp.float32)
          + jnp.dot(a2_ref[...], x2, preferred_element_type=jnp.float32)
          + b1_ref[...])
    a = jnp.maximum(h1, 0.01 * h1)
    h2 = jnp.dot(w2_ref[...], a, preferred_element_type=jnp.float32) + b2_ref[...]
    nl = _BLC // 32
    feat_ref[:, 0:1, :] = f1_ref[...].reshape(1, nl, 4096).transpose(1, 0, 2)
    feat_ref[:, 1:2, :] = f2_ref[...].reshape(1, nl, 4096).transpose(1, 0, 2)
    feat_ref[:, 2:7, :] = x2[20:25].reshape(5, nl, 4096).transpose(1, 0, 2)
    feat_ref[:, 7:20, :] = (h2[0:13] + x1[0:13]).reshape(13, nl, 4096).transpose(1, 0, 2)
    feat_ref[:, 20:40, :] = (h2[13:33] + x2[0:20]).reshape(20, nl, 4096).transpose(1, 0, 2)
    wk_ref[...] = x1[0:3].reshape(3, nl, 4096)
    dr_ref[...] = x1[3:13].reshape(10, nl, 4096)
    tr_ref[...] = x2[0:20].reshape(20, nl, 4096)


def _tc_call(x1, x2, f1t, f2t, a1, a2, w2, b1c, b2c):
    nch = x1.shape[0]
    grid = (nch // _BLC,)
    nl = _BLC // 32
    full = lambda a: pl.BlockSpec(a.shape, lambda i: (0,) * a.ndim)
    out3 = lambda d: pl.BlockSpec((d, nl, 4096), lambda i: (0, i, 0))
    feat_spec = pl.BlockSpec((nl, 40, 4096), lambda i: (i, 0, 0))
    return pl.pallas_call(
        _tc_body,
        grid=grid,
        in_specs=[
            pl.BlockSpec((_BLC, 16, _SUB), lambda i: (i, 0, 0)),
            pl.BlockSpec((_BLC, 32, _SUB), lambda i: (i, 0, 0)),
            pl.BlockSpec((nl, 4096), lambda i: (i, 0)),
            pl.BlockSpec((nl, 4096), lambda i: (i, 0)),
            full(a1), full(a2), full(w2), full(b1c), full(b2c),
        ],
        out_specs=[feat_spec, out3(3), out3(10), out3(20)],
        out_shape=[
            jax.ShapeDtypeStruct((200, 40, 4096), jnp.float32),
            jax.ShapeDtypeStruct((3, 200, 4096), jnp.float32),
            jax.ShapeDtypeStruct((10, 200, 4096), jnp.float32),
            jax.ShapeDtypeStruct((20, 200, 4096), jnp.float32),
        ],
    )(x1, x2, f1t, f2t, a1, a2, w2, b1c, b2c)


def kernel(links, hw_tab, wk_tab, dt_tab, tm_tab, W1, b1, W2, b2, args):
    bb, ll, _ = links.shape
    n = bb * ll

    i0 = links[:, :, 0].astype(jnp.int32).T.reshape(n)
    i3 = links[:, :, 3].astype(jnp.int32).T.reshape(n)
    i4 = links[:, :, 4].astype(jnp.int32).T.reshape(n)
    i5 = links[:, :, 5].astype(jnp.int32).T.reshape(n)
    f1t = links[:, :, 1].T
    f2t = links[:, :, 2].T
    fdi = i3 * 367 + i4
    fti = i0 * 1441 + i5

    hw_z = hw_tab.at[0].set(0.0)
    nw, nd = wk_tab.shape[0], dt_tab.shape[0]
    nh, nt = hw_tab.shape[0], tm_tab.shape[0]
    fd_tab = jnp.concatenate([
        jnp.broadcast_to(wk_tab[:, None, :], (nw, nd, 3)),
        jnp.broadcast_to(dt_tab[None, :, :], (nw, nd, 10)),
        jnp.zeros((nw, nd, 3), jnp.float32),
    ], axis=-1).reshape(nw * nd, 16)
    ft_tab = jnp.concatenate([
        jnp.broadcast_to(tm_tab[None, :, :], (nh, nt, 20)),
        jnp.broadcast_to(hw_z[:, None, :], (nh, nt, 5)),
        jnp.zeros((nh, nt, 7), jnp.float32),
    ], axis=-1).reshape(nh * nt, 32)

    x1, x2 = _sc_gather_call(fdi, fti, fd_tab, ft_tab)

    a1 = jnp.concatenate([W1[:, 0:13], jnp.zeros((33, 3), jnp.float32)], 1)
    a2 = jnp.concatenate([W1[:, 13:33], jnp.zeros((33, 12), jnp.float32)], 1)
    featt, wkt, drt, trt = _tc_call(x1, x2, f1t, f2t, a1, a2, W2,
                                    b1.reshape(33, 1), b2.reshape(33, 1))
    return (featt.transpose(2, 0, 1),
            (wkt.transpose(2, 1, 0), drt.transpose(2, 1, 0),
             trt.transpose(2, 1, 0)))


# speedup vs baseline: 11.2762x; 1.4681x over previous
import functools

import jax
import jax.numpy as jnp
from jax import lax
from jax.experimental import pallas as pl
from jax.experimental.pallas import tpu as pltpu
from jax.experimental.pallas import tpu_sc as plsc

_NW = 32
_SUB = 128
_LANES = 16
_BLC = 256


def _sc_gather_call(fdi, fti, fd_tab, ft_tab):
    n = fdi.shape[0]
    nch = n // _SUB
    rows_w = n // _NW
    ch_w = rows_w // _SUB
    mesh = plsc.VectorSubcoreMesh(core_axis_name="c", subcore_axis_name="s")

    @functools.partial(
        pl.kernel,
        out_type=jax.ShapeDtypeStruct((nch, 48, _SUB), jnp.float32),
        mesh=mesh,
        compiler_params=pltpu.CompilerParams(use_tc_tiling_on_sc=False,
                                             needs_layout_passes=False),
        scratch_types=[
            pltpu.VMEM((rows_w,), jnp.int32),
            pltpu.VMEM((rows_w,), jnp.int32),
            pltpu.VMEM((2, _SUB, 16), jnp.float32),
            pltpu.VMEM((2, _SUB, 32), jnp.float32),
            pltpu.VMEM((2, 48, _SUB), jnp.float32),
            pltpu.SemaphoreType.DMA,
            pltpu.SemaphoreType.DMA,
            pltpu.SemaphoreType.DMA,
            pltpu.SemaphoreType.DMA,
            pltpu.SemaphoreType.DMA,
            pltpu.SemaphoreType.DMA,
            pltpu.SemaphoreType.DMA,
        ],
    )
    def sc_kernel(fdi_h, fti_h, fd_h, ft_h, x_h,
                  fdi_v, fti_v, bfd, bft, tb,
                  semi, semgf0, semgf1, semgt0, semgt1, semw0, semw1):
        iota = lax.iota(jnp.int32, _LANES)
        wid = lax.axis_index("s") * 2 + lax.axis_index("c")
        base_c = wid * ch_w
        semgf = (semgf0, semgf1)
        semgt = (semgt0, semgt1)
        semw = (semw0, semw1)

        pltpu.async_copy(fdi_h.at[pl.ds(wid * rows_w, rows_w)], fdi_v, semi).wait()
        pltpu.async_copy(fti_h.at[pl.ds(wid * rows_w, rows_w)], fti_v, semi).wait()

        def issue(ci, b):
            s = pl.ds(ci * _SUB, _SUB)
            pltpu.async_copy(fd_h.at[fdi_v.at[s]], bfd.at[b], semgf[b])
            pltpu.async_copy(ft_h.at[fti_v.at[s]], bft.at[b], semgt[b])

        def wait_gather(ci, b):
            s = pl.ds(ci * _SUB, _SUB)
            pltpu.make_async_copy(fd_h.at[fdi_v.at[s]], bfd.at[b], semgf[b]).wait()
            pltpu.make_async_copy(ft_h.at[fti_v.at[s]], bft.at[b], semgt[b]).wait()

        def wait_wb(ci, b):
            pltpu.make_async_copy(tb.at[b], x_h.at[base_c + ci], semw[b]).wait()

        zero = jnp.zeros((_LANES,), jnp.float32)
        for b in range(2):
            for f in list(range(13, 16)) + list(range(41, 48)):
                for g in range(_SUB // _LANES):
                    tb[b, f, pl.ds(g * _LANES, _LANES)] = zero

        issue(0, 0)
        issue(1, 1)

        def pair(pi, carry):
            for b in range(2):
                ci = pi * 2 + b
                wait_gather(ci, b)

                @pl.when(ci >= 2)
                def _():
                    wait_wb(ci - 2, b)

                for f in range(13):
                    cols = jnp.full((_LANES,), f, jnp.int32)
                    for g in range(_SUB // _LANES):
                        rows = iota + (g * _LANES)
                        tb[b, f, pl.ds(g * _LANES, _LANES)] = plsc.load_gather(
                            bfd.at[b], [rows, cols])
                for f in range(25):
                    cols = jnp.full((_LANES,), f, jnp.int32)
                    for g in range(_SUB // _LANES):
                        rows = iota + (g * _LANES)
                        tb[b, 16 + f, pl.ds(g * _LANES, _LANES)] = plsc.load_gather(
                            bft.at[b], [rows, cols])

                pltpu.async_copy(tb.at[b], x_h.at[base_c + ci], semw[b])

                @pl.when(ci + 2 < ch_w)
                def _():
                    issue(ci + 2, b)
            return carry

        lax.fori_loop(0, ch_w // 2, pair, 0)
        wait_wb(ch_w - 2, 0)
        wait_wb(ch_w - 1, 1)

    return sc_kernel(fdi, fti, fd_tab, ft_tab)


def _tc_body(x_ref, f1_ref, f2_ref, a1_ref, a2_ref, w2_ref,
             b1_ref, b2_ref, feat_ref, wk_ref, dr_ref, tr_ref):
    r = _BLC * _SUB
    x = x_ref[...].transpose(1, 0, 2).reshape(48, r)
    x1 = x[0:16]
    x2 = x[16:48]
    h1 = (jnp.dot(a1_ref[...], x1, preferred_element_type=jnp.float32)
          + jnp.dot(a2_ref[...], x2, preferred_element_type=jnp.float32)
          + b1_ref[...])
    a = jnp.maximum(h1, 0.01 * h1)
    h2 = jnp.dot(w2_ref[...], a, preferred_element_type=jnp.float32) + b2_ref[...]
    nl = _BLC // 32
    feat_ref[:, 0:1, :] = f1_ref[...].reshape(1, nl, 4096).transpose(1, 0, 2)
    feat_ref[:, 1:2, :] = f2_ref[...].reshape(1, nl, 4096).transpose(1, 0, 2)
    feat_ref[:, 2:7, :] = x2[20:25].reshape(5, nl, 4096).transpose(1, 0, 2)
    feat_ref[:, 7:20, :] = (h2[0:13] + x1[0:13]).reshape(13, nl, 4096).transpose(1, 0, 2)
    feat_ref[:, 20:40, :] = (h2[13:33] + x2[0:20]).reshape(20, nl, 4096).transpose(1, 0, 2)
    wk_ref[...] = x1[0:3].reshape(3, nl, 4096)
    dr_ref[...] = x1[3:13].reshape(10, nl, 4096)
    tr_ref[...] = x2[0:20].reshape(20, nl, 4096)


def _tc_call(x, f1t, f2t, a1, a2, w2, b1c, b2c):
    nch = x.shape[0]
    grid = (nch // _BLC,)
    nl = _BLC // 32
    full = lambda a: pl.BlockSpec(a.shape, lambda i: (0,) * a.ndim)
    out3 = lambda d: pl.BlockSpec((d, nl, 4096), lambda i: (0, i, 0))
    feat_spec = pl.BlockSpec((nl, 40, 4096), lambda i: (i, 0, 0))
    return pl.pallas_call(
        _tc_body,
        grid=grid,
        in_specs=[
            pl.BlockSpec((_BLC, 48, _SUB), lambda i: (i, 0, 0)),
            pl.BlockSpec((nl, 4096), lambda i: (i, 0)),
            pl.BlockSpec((nl, 4096), lambda i: (i, 0)),
            full(a1), full(a2), full(w2), full(b1c), full(b2c),
        ],
        out_specs=[feat_spec, out3(3), out3(10), out3(20)],
        out_shape=[
            jax.ShapeDtypeStruct((200, 40, 4096), jnp.float32),
            jax.ShapeDtypeStruct((3, 200, 4096), jnp.float32),
            jax.ShapeDtypeStruct((10, 200, 4096), jnp.float32),
            jax.ShapeDtypeStruct((20, 200, 4096), jnp.float32),
        ],
    )(x, f1t, f2t, a1, a2, w2, b1c, b2c)


def kernel(links, hw_tab, wk_tab, dt_tab, tm_tab, W1, b1, W2, b2, args):
    bb, ll, _ = links.shape
    n = bb * ll

    i0 = links[:, :, 0].astype(jnp.int32).T.reshape(n)
    i3 = links[:, :, 3].astype(jnp.int32).T.reshape(n)
    i4 = links[:, :, 4].astype(jnp.int32).T.reshape(n)
    i5 = links[:, :, 5].astype(jnp.int32).T.reshape(n)
    f1t = links[:, :, 1].T
    f2t = links[:, :, 2].T
    fdi = i3 * 367 + i4
    fti = i0 * 1441 + i5

    hw_z = hw_tab.at[0].set(0.0)
    nw, nd = wk_tab.shape[0], dt_tab.shape[0]
    nh, nt = hw_tab.shape[0], tm_tab.shape[0]
    fd_tab = jnp.concatenate([
        jnp.broadcast_to(wk_tab[:, None, :], (nw, nd, 3)),
        jnp.broadcast_to(dt_tab[None, :, :], (nw, nd, 10)),
        jnp.zeros((nw, nd, 3), jnp.float32),
    ], axis=-1).reshape(nw * nd, 16)
    ft_tab = jnp.concatenate([
        jnp.broadcast_to(tm_tab[None, :, :], (nh, nt, 20)),
        jnp.broadcast_to(hw_z[:, None, :], (nh, nt, 5)),
        jnp.zeros((nh, nt, 7), jnp.float32),
    ], axis=-1).reshape(nh * nt, 32)

    x = _sc_gather_call(fdi, fti, fd_tab, ft_tab)

    a1 = jnp.concatenate([W1[:, 0:13], jnp.zeros((33, 3), jnp.float32)], 1)
    a2 = jnp.concatenate([W1[:, 13:33], jnp.zeros((33, 12), jnp.float32)], 1)
    featt, wkt, drt, trt = _tc_call(x, f1t, f2t, a1, a2, W2,
                                    b1.reshape(33, 1), b2.reshape(33, 1))
    return (featt.transpose(2, 0, 1),
            (wkt.transpose(2, 1, 0), drt.transpose(2, 1, 0),
             trt.transpose(2, 1, 0)))


# speedup vs baseline: 14.7815x; 1.3109x over previous
import functools

import jax
import jax.numpy as jnp
from jax import lax
from jax.experimental import pallas as pl
from jax.experimental.pallas import tpu as pltpu
from jax.experimental.pallas import tpu_sc as plsc

_NW = 32
_SUB = 128
_LANES = 16
_BLC = 256


def _sc_gather_call(fdi, fti, fd_tab, ft_tab):
    n = fdi.shape[0]
    nch = n // _SUB
    rows_w = n // _NW
    ch_w = rows_w // _SUB
    mesh = plsc.VectorSubcoreMesh(core_axis_name="c", subcore_axis_name="s")

    @functools.partial(
        pl.kernel,
        out_type=jax.ShapeDtypeStruct((nch, 48, _SUB), jnp.float32),
        mesh=mesh,
        compiler_params=pltpu.CompilerParams(use_tc_tiling_on_sc=False,
                                             needs_layout_passes=False),
        scratch_types=[
            pltpu.VMEM((rows_w,), jnp.int32),
            pltpu.VMEM((rows_w,), jnp.int32),
            pltpu.VMEM((2, _SUB, 16), jnp.float32),
            pltpu.VMEM((2, _SUB, 32), jnp.float32),
            pltpu.VMEM((2, 48, _SUB), jnp.float32),
            pltpu.SemaphoreType.DMA,
            pltpu.SemaphoreType.DMA,
            pltpu.SemaphoreType.DMA,
            pltpu.SemaphoreType.DMA,
            pltpu.SemaphoreType.DMA,
            pltpu.SemaphoreType.DMA,
            pltpu.SemaphoreType.DMA,
        ],
    )
    def sc_kernel(fdi_h, fti_h, fd_h, ft_h, x_h,
                  fdi_v, fti_v, bfd, bft, tb,
                  semi, semgf0, semgf1, semgt0, semgt1, semw0, semw1):
        iota = lax.iota(jnp.int32, _LANES)
        rows_l = [iota + (g * _LANES) for g in range(_SUB // _LANES)]
        wid = lax.axis_index("s") * 2 + lax.axis_index("c")
        base_c = wid * ch_w
        semgf = (semgf0, semgf1)
        semgt = (semgt0, semgt1)
        semw = (semw0, semw1)

        pltpu.async_copy(fdi_h.at[pl.ds(wid * rows_w, rows_w)], fdi_v, semi).wait()
        pltpu.async_copy(fti_h.at[pl.ds(wid * rows_w, rows_w)], fti_v, semi).wait()

        def issue(ci, b):
            s = pl.ds(ci * _SUB, _SUB)
            pltpu.async_copy(fd_h.at[fdi_v.at[s]], bfd.at[b], semgf[b])
            pltpu.async_copy(ft_h.at[fti_v.at[s]], bft.at[b], semgt[b])

        def wait_gather(ci, b):
            s = pl.ds(ci * _SUB, _SUB)
            pltpu.make_async_copy(fd_h.at[fdi_v.at[s]], bfd.at[b], semgf[b]).wait()
            pltpu.make_async_copy(ft_h.at[fti_v.at[s]], bft.at[b], semgt[b]).wait()

        def wait_wb(ci, b):
            pltpu.make_async_copy(tb.at[b], x_h.at[base_c + ci], semw[b]).wait()

        zero = jnp.zeros((_LANES,), jnp.float32)
        for b in range(2):
            for f in list(range(13, 16)) + list(range(41, 48)):
                for g in range(_SUB // _LANES):
                    tb[b, f, pl.ds(g * _LANES, _LANES)] = zero

        issue(0, 0)
        issue(1, 1)

        def pair(pi, carry):
            for b in range(2):
                ci = pi * 2 + b
                wait_gather(ci, b)

                @pl.when(ci >= 2)
                def _():
                    wait_wb(ci - 2, b)

                for f in range(13):
                    cols = jnp.full((_LANES,), f, jnp.int32)
                    vs = [plsc.load_gather(bfd.at[b], [rows_l[g], cols])
                          for g in range(_SUB // _LANES)]
                    for g in range(_SUB // _LANES):
                        tb[b, f, pl.ds(g * _LANES, _LANES)] = vs[g]
                for f in range(25):
                    cols = jnp.full((_LANES,), f, jnp.int32)
                    vs = [plsc.load_gather(bft.at[b], [rows_l[g], cols])
                          for g in range(_SUB // _LANES)]
                    for g in range(_SUB // _LANES):
                        tb[b, 16 + f, pl.ds(g * _LANES, _LANES)] = vs[g]

                pltpu.async_copy(tb.at[b], x_h.at[base_c + ci], semw[b])

                @pl.when(ci + 2 < ch_w)
                def _():
                    issue(ci + 2, b)
            return carry

        lax.fori_loop(0, ch_w // 2, pair, 0)
        wait_wb(ch_w - 2, 0)
        wait_wb(ch_w - 1, 1)

    return sc_kernel(fdi, fti, fd_tab, ft_tab)


def _tc_body(x_ref, f1_ref, f2_ref, a1_ref, a2_ref, w2_ref,
             b1_ref, b2_ref, feat_ref, wk_ref, dr_ref, tr_ref):
    r = _BLC * _SUB
    x = x_ref[...].transpose(1, 0, 2).reshape(48, r)
    x1 = x[0:16]
    x2 = x[16:48]
    h1 = (jnp.dot(a1_ref[...], x1, preferred_element_type=jnp.float32)
          + jnp.dot(a2_ref[...], x2, preferred_element_type=jnp.float32)
          + b1_ref[...])
    a = jnp.maximum(h1, 0.01 * h1)
    h2 = jnp.dot(w2_ref[...], a, preferred_element_type=jnp.float32) + b2_ref[...]
    nl = _BLC // 32
    feat_ref[:, 0:1, :] = f1_ref[...].reshape(1, nl, 4096).transpose(1, 0, 2)
    feat_ref[:, 1:2, :] = f2_ref[...].reshape(1, nl, 4096).transpose(1, 0, 2)
    feat_ref[:, 2:7, :] = x2[20:25].reshape(5, nl, 4096).transpose(1, 0, 2)
    feat_ref[:, 7:20, :] = (h2[0:13] + x1[0:13]).reshape(13, nl, 4096).transpose(1, 0, 2)
    feat_ref[:, 20:40, :] = (h2[13:33] + x2[0:20]).reshape(20, nl, 4096).transpose(1, 0, 2)
    wk_ref[...] = x1[0:3].reshape(3, nl, 4096)
    dr_ref[...] = x1[3:13].reshape(10, nl, 4096)
    tr_ref[...] = x2[0:20].reshape(20, nl, 4096)


def _tc_call(x, f1t, f2t, a1, a2, w2, b1c, b2c):
    nch = x.shape[0]
    grid = (nch // _BLC,)
    nl = _BLC // 32
    full = lambda a: pl.BlockSpec(a.shape, lambda i: (0,) * a.ndim)
    out3 = lambda d: pl.BlockSpec((d, nl, 4096), lambda i: (0, i, 0))
    feat_spec = pl.BlockSpec((nl, 40, 4096), lambda i: (i, 0, 0))
    return pl.pallas_call(
        _tc_body,
        grid=grid,
        in_specs=[
            pl.BlockSpec((_BLC, 48, _SUB), lambda i: (i, 0, 0)),
            pl.BlockSpec((nl, 4096), lambda i: (i, 0)),
            pl.BlockSpec((nl, 4096), lambda i: (i, 0)),
            full(a1), full(a2), full(w2), full(b1c), full(b2c),
        ],
        out_specs=[feat_spec, out3(3), out3(10), out3(20)],
        out_shape=[
            jax.ShapeDtypeStruct((200, 40, 4096), jnp.float32),
            jax.ShapeDtypeStruct((3, 200, 4096), jnp.float32),
            jax.ShapeDtypeStruct((10, 200, 4096), jnp.float32),
            jax.ShapeDtypeStruct((20, 200, 4096), jnp.float32),
        ],
    )(x, f1t, f2t, a1, a2, w2, b1c, b2c)


def kernel(links, hw_tab, wk_tab, dt_tab, tm_tab, W1, b1, W2, b2, args):
    bb, ll, _ = links.shape
    n = bb * ll

    i0 = links[:, :, 0].astype(jnp.int32).T.reshape(n)
    i3 = links[:, :, 3].astype(jnp.int32).T.reshape(n)
    i4 = links[:, :, 4].astype(jnp.int32).T.reshape(n)
    i5 = links[:, :, 5].astype(jnp.int32).T.reshape(n)
    f1t = links[:, :, 1].T
    f2t = links[:, :, 2].T
    fdi = i3 * 367 + i4
    fti = i0 * 1441 + i5

    hw_z = hw_tab.at[0].set(0.0)
    nw, nd = wk_tab.shape[0], dt_tab.shape[0]
    nh, nt = hw_tab.shape[0], tm_tab.shape[0]
    fd_tab = jnp.concatenate([
        jnp.broadcast_to(wk_tab[:, None, :], (nw, nd, 3)),
        jnp.broadcast_to(dt_tab[None, :, :], (nw, nd, 10)),
        jnp.zeros((nw, nd, 3), jnp.float32),
    ], axis=-1).reshape(nw * nd, 16)
    ft_tab = jnp.concatenate([
        jnp.broadcast_to(tm_tab[None, :, :], (nh, nt, 20)),
        jnp.broadcast_to(hw_z[:, None, :], (nh, nt, 5)),
        jnp.zeros((nh, nt, 7), jnp.float32),
    ], axis=-1).reshape(nh * nt, 32)

    x = _sc_gather_call(fdi, fti, fd_tab, ft_tab)

    a1 = jnp.concatenate([W1[:, 0:13], jnp.zeros((33, 3), jnp.float32)], 1)
    a2 = jnp.concatenate([W1[:, 13:33], jnp.zeros((33, 12), jnp.float32)], 1)
    featt, wkt, drt, trt = _tc_call(x, f1t, f2t, a1, a2, W2,
                                    b1.reshape(33, 1), b2.reshape(33, 1))
    return (featt.transpose(2, 0, 1),
            (wkt.transpose(2, 1, 0), drt.transpose(2, 1, 0),
             trt.transpose(2, 1, 0)))
